# baseline (device time: 37445 ns/iter reference)
import jax
import jax.numpy as jnp
from jax import lax
from jax.experimental import pallas as pl
from jax.experimental.pallas import tpu as pltpu

N_DEV = 16
M = 512
D = 512
PM = M // 4
CM = PM // 4
EPS = 1e-6


def kernel(partial, resid, gamma):
    partial = partial.reshape(M, D)

    def body(partial_ref, resid_ref, gamma_ref, out_ref,
             p1_buf, red_buf, p2_buf,
             send1, recv1, send2, recv2, send3, recv3, send4, recv4):
        i = lax.axis_index("i")
        z = i // 4
        w = i % 4

        rdmas1 = []
        for d in range(1, 4):
            pw = lax.rem(w + d, 4)
            peer = 4 * z + pw
            rdma = pltpu.make_async_remote_copy(
                src_ref=partial_ref.at[pl.ds(pw * PM, PM), :],
                dst_ref=p1_buf.at[d],
                send_sem=send1.at[d],
                recv_sem=recv1.at[d],
                device_id=(peer,),
                device_id_type=pl.DeviceIdType.MESH,
            )
            rdma.start()
            rdmas1.append(rdma)

        for rdma in rdmas1:
            rdma.wait_recv()
        y128 = partial_ref[pl.ds(w * PM, PM), :]
        for d in range(1, 4):
            y128 = y128 + p1_buf[d]
        red_buf[...] = y128

        rdmas2 = []
        for d in range(1, 4):
            pz = lax.rem(z + d, 4)
            peer = 4 * pz + w
            rdma = pltpu.make_async_remote_copy(
                src_ref=red_buf.at[pl.ds(pz * CM, CM), :],
                dst_ref=p2_buf.at[d],
                send_sem=send2.at[d],
                recv_sem=recv2.at[d],
                device_id=(peer,),
                device_id_type=pl.DeviceIdType.MESH,
            )
            rdma.start()
            rdmas2.append(rdma)

        for rdma in rdmas1:
            rdma.wait_send()
        for rdma in rdmas2:
            rdma.wait_recv()

        row0 = w * PM + z * CM
        y = red_buf[pl.ds(z * CM, CM), :]
        for d in range(1, 4):
            y = y + p2_buf[d]
        y = y + resid_ref[pl.ds(row0, CM), :]
        rms = jnp.sqrt(jnp.mean(y * y, axis=-1, keepdims=True) + EPS)
        out_ref[pl.ds(row0, CM), :] = y / rms * gamma_ref[...]

        rdmas3 = []
        for d in range(1, 4):
            pz = lax.rem(z + d, 4)
            peer = 4 * pz + w
            rdma = pltpu.make_async_remote_copy(
                src_ref=out_ref.at[pl.ds(row0, CM), :],
                dst_ref=out_ref.at[pl.ds(row0, CM), :],
                send_sem=send3.at[d],
                recv_sem=recv3.at[d],
                device_id=(peer,),
                device_id_type=pl.DeviceIdType.MESH,
            )
            rdma.start()
            rdmas3.append(rdma)

        for rdma in rdmas2:
            rdma.wait_send()
        for rdma in rdmas3:
            rdma.wait_recv()

        rdmas4 = []
        for d in range(1, 4):
            pw = lax.rem(w + d, 4)
            peer = 4 * z + pw
            rdma = pltpu.make_async_remote_copy(
                src_ref=out_ref.at[pl.ds(w * PM, PM), :],
                dst_ref=out_ref.at[pl.ds(w * PM, PM), :],
                send_sem=send4.at[d],
                recv_sem=recv4.at[d],
                device_id=(peer,),
                device_id_type=pl.DeviceIdType.MESH,
            )
            rdma.start()
            rdmas4.append(rdma)

        for rdma in rdmas3:
            rdma.wait_send()
        for rdma in rdmas4:
            rdma.wait_recv()
            rdma.wait_send()

    return pl.pallas_call(
        body,
        out_shape=jax.ShapeDtypeStruct((M, D), jnp.float32),
        in_specs=[
            pl.BlockSpec(memory_space=pltpu.VMEM),
            pl.BlockSpec(memory_space=pltpu.VMEM),
            pl.BlockSpec(memory_space=pltpu.VMEM),
        ],
        out_specs=pl.BlockSpec(memory_space=pltpu.VMEM),
        scratch_shapes=[
            pltpu.VMEM((4, PM, D), jnp.float32),
            pltpu.VMEM((PM, D), jnp.float32),
            pltpu.VMEM((4, CM, D), jnp.float32),
            pltpu.SemaphoreType.DMA((4,)),
            pltpu.SemaphoreType.DMA((4,)),
            pltpu.SemaphoreType.DMA((4,)),
            pltpu.SemaphoreType.DMA((4,)),
            pltpu.SemaphoreType.DMA((4,)),
            pltpu.SemaphoreType.DMA((4,)),
            pltpu.SemaphoreType.DMA((4,)),
            pltpu.SemaphoreType.DMA((4,)),
        ],
    )(partial, resid, gamma)


# device time: 27985 ns/iter; 1.3380x vs baseline; 1.3380x over previous
import jax
import jax.numpy as jnp
from jax import lax
from jax.experimental import pallas as pl
from jax.experimental.pallas import tpu as pltpu

N_DEV = 16
M = 512
D = 512
PM = M // 4
CM = PM // 4
EPS = 1e-6

_FAR_FIRST = {0: (3, 2, 1), 1: (2, 1, 3), 2: (2, 1, 3), 3: (1, 2, 3)}
_NEAR_FIRST = {0: (3, 2, 1), 1: (1, 3, 2), 2: (1, 3, 2), 3: (1, 2, 3)}


def kernel(partial, resid, gamma):
    def body(partial_ref, resid_ref, gamma_ref, out_ref,
             p1_buf, zbuf, p2_buf,
             s1, r1, s2, r2, s3, r3, s4, r4):
        i = lax.axis_index("i")
        z = i // 4
        w = i % 4
        row0 = w * PM + z * CM

        def plane_peer(d):
            return 4 * z + lax.rem(w + d, 4)

        def col_peer(j):
            return 4 * lax.rem(z + j, 4) + w

        bsem = pltpu.get_barrier_semaphore()
        for d in range(1, 4):
            pl.semaphore_signal(bsem, inc=1, device_id=(plane_peer(d),),
                                device_id_type=pl.DeviceIdType.MESH)
            pl.semaphore_signal(bsem, inc=1, device_id=(col_peer(d),),
                                device_id_type=pl.DeviceIdType.MESH)
        pl.semaphore_wait(bsem, 6)

        p1 = {}
        for j in range(4):
            sub = lax.rem(z + j, 4)
            for d in range(1, 4):
                pw = lax.rem(w + d, 4)
                p1[(d, j)] = pltpu.make_async_remote_copy(
                    src_ref=partial_ref.at[0, pl.ds(pw * PM + sub * CM, CM), :],
                    dst_ref=p1_buf.at[d - 1, j],
                    send_sem=s1.at[d - 1, j],
                    recv_sem=r1.at[d - 1, j],
                    device_id=(plane_peer(d),),
                    device_id_type=pl.DeviceIdType.MESH,
                )
        rdmas2 = {}
        for j in range(1, 4):
            rdmas2[j] = pltpu.make_async_remote_copy(
                src_ref=zbuf.at[j - 1],
                dst_ref=p2_buf.at[j - 1],
                send_sem=s2.at[j - 1],
                recv_sem=r2.at[j - 1],
                device_id=(col_peer(j),),
                device_id_type=pl.DeviceIdType.MESH,
            )
        rdmas3 = {}
        for j in range(1, 4):
            rdmas3[j] = pltpu.make_async_remote_copy(
                src_ref=out_ref.at[pl.ds(row0, CM), :],
                dst_ref=out_ref.at[pl.ds(row0, CM), :],
                send_sem=s3.at[j - 1],
                recv_sem=r3.at[j - 1],
                device_id=(col_peer(j),),
                device_id_type=pl.DeviceIdType.MESH,
            )
        p4 = {}
        for d in range(1, 4):
            p4[(d, 0)] = pltpu.make_async_remote_copy(
                src_ref=out_ref.at[pl.ds(row0, CM), :],
                dst_ref=out_ref.at[pl.ds(row0, CM), :],
                send_sem=s4.at[d - 1, 0],
                recv_sem=r4.at[d - 1, 0],
                device_id=(plane_peer(d),),
                device_id_type=pl.DeviceIdType.MESH,
            )
        for j in range(1, 4):
            rowj = w * PM + lax.rem(z - j + 4, 4) * CM
            for d in range(1, 4):
                p4[(d, j)] = pltpu.make_async_remote_copy(
                    src_ref=out_ref.at[pl.ds(rowj, CM), :],
                    dst_ref=out_ref.at[pl.ds(rowj, CM), :],
                    send_sem=s4.at[d - 1, j],
                    recv_sem=r4.at[d - 1, j],
                    device_id=(plane_peer(d),),
                    device_id_type=pl.DeviceIdType.MESH,
                )

        for zk in range(4):
            @pl.when(z == zk)
            def _(zk=zk):
                order = _FAR_FIRST[zk]
                for j in order + (0,):
                    for d in range(1, 4):
                        p1[(d, j)].start()
                for j in order:
                    for d in range(1, 4):
                        p1[(d, j)].wait_recv()
                    sub = lax.rem(z + j, 4)
                    acc = partial_ref[0, pl.ds(w * PM + sub * CM, CM), :]
                    for d in range(1, 4):
                        acc = acc + p1_buf[d - 1, j]
                    zbuf[j - 1] = acc
                    rdmas2[j].start()

        for d in range(1, 4):
            p1[(d, 0)].wait_recv()
        y = partial_ref[0, pl.ds(row0, CM), :]
        for d in range(1, 4):
            y = y + p1_buf[d - 1, 0]
        for j in range(1, 4):
            rdmas2[j].wait_recv()
            y = y + p2_buf[j - 1]
        y = y + resid_ref[pl.ds(row0, CM), :]
        rms = jnp.sqrt(jnp.mean(y * y, axis=-1, keepdims=True) + EPS)
        out_ref[pl.ds(row0, CM), :] = y / rms * gamma_ref[...]

        for zk in range(4):
            @pl.when(z == zk)
            def _(zk=zk):
                for j in _FAR_FIRST[zk]:
                    rdmas3[j].start()
                for d in range(1, 4):
                    p4[(d, 0)].start()
                for j in _NEAR_FIRST[zk]:
                    rdmas3[j].wait_recv()
                    for d in range(1, 4):
                        p4[(d, j)].start()

        for rdma in p4.values():
            rdma.wait_recv()

        for rdma in p1.values():
            rdma.wait_send()
        for rdma in rdmas2.values():
            rdma.wait_send()
        for rdma in rdmas3.values():
            rdma.wait_send()
        for rdma in p4.values():
            rdma.wait_send()

    return pl.pallas_call(
        body,
        out_shape=jax.ShapeDtypeStruct((M, D), jnp.float32),
        in_specs=[
            pl.BlockSpec(memory_space=pltpu.VMEM),
            pl.BlockSpec(memory_space=pltpu.VMEM),
            pl.BlockSpec(memory_space=pltpu.VMEM),
        ],
        out_specs=pl.BlockSpec(memory_space=pltpu.VMEM),
        scratch_shapes=[
            pltpu.VMEM((3, 4, CM, D), jnp.float32),
            pltpu.VMEM((3, CM, D), jnp.float32),
            pltpu.VMEM((3, CM, D), jnp.float32),
            pltpu.SemaphoreType.DMA((3, 4)),
            pltpu.SemaphoreType.DMA((3, 4)),
            pltpu.SemaphoreType.DMA((3,)),
            pltpu.SemaphoreType.DMA((3,)),
            pltpu.SemaphoreType.DMA((3,)),
            pltpu.SemaphoreType.DMA((3,)),
            pltpu.SemaphoreType.DMA((3, 4)),
            pltpu.SemaphoreType.DMA((3, 4)),
        ],
        compiler_params=pltpu.CompilerParams(collective_id=0),
    )(partial, resid, gamma)
